# baseline (speedup 1.0000x reference)
import functools

import jax
import jax.numpy as jnp
from jax import lax
from jax.experimental import pallas as pl
from jax.experimental.pallas import tpu as pltpu
from jax.experimental.pallas import tpu_sc as plsc

_VOCAB = 1_000_000
_SIZE = 64
_BATCH = 4096
_SEQ = 200
_NC = 2
_NS = 16
_NW = _NC * _NS
_ROWS = _BATCH * _SEQ
_RPW = _ROWS // _NW
_CH = _SEQ
_NCHUNK = _RPW // _CH
_NBUF = 2
_NOUTER = _NCHUNK // _NBUF
_LANES = 16
_NVEC = _SIZE // _LANES
_G0 = 128
_G1 = _CH - _G0


def _positional_rows():
    pos = jnp.arange(_SEQ, dtype=jnp.float32)[:, None]
    period = jnp.power(10000.0, 2.0 * jnp.arange(_SIZE // 2, dtype=jnp.float32) / _SIZE)
    sin = jnp.sin(pos / period[None, :])
    cos = jnp.cos(pos / period[None, :])
    pe = jnp.zeros((_SEQ, _SIZE), dtype=jnp.float32)
    pe = pe.at[:, 0::2].set(sin)
    pe = pe.at[:, 1::2].set(cos)
    return pe


_mesh = plsc.VectorSubcoreMesh(core_axis_name="c", subcore_axis_name="s")


@functools.partial(
    pl.kernel,
    out_type=jax.ShapeDtypeStruct((_ROWS, _SIZE), jnp.float32),
    mesh=_mesh,
    scratch_types=[
        pltpu.VMEM((_RPW,), jnp.int32),
        pltpu.VMEM((_SEQ, _SIZE), jnp.float32),
        pltpu.VMEM((_NBUF, _CH, _SIZE), jnp.float32),
        pltpu.VMEM((_NBUF, _CH, _SIZE), jnp.float32),
        pltpu.SemaphoreType.DMA,
        pltpu.SemaphoreType.DMA,
        pltpu.SemaphoreType.DMA,
        pltpu.SemaphoreType.DMA,
    ],
    compiler_params=pltpu.CompilerParams(use_tc_tiling_on_sc=False),
)
def _emb_kernel(table_hbm, idx_hbm, pe_hbm, out_hbm,
                idx_v, pe_v, gbuf, obuf, gsem0, gsem1, ssem0, ssem1):
    wid = lax.axis_index("s") * _NC + lax.axis_index("c")
    base = wid * _RPW
    gsems = (gsem0, gsem1)
    ssems = (ssem0, ssem1)

    pltpu.sync_copy(idx_hbm.at[pl.ds(base, _RPW)], idx_v)
    pltpu.sync_copy(pe_hbm, pe_v)

    def gather_descs(i, b):
        c0 = i * _CH
        return (
            pltpu.make_async_copy(
                table_hbm.at[idx_v.at[pl.ds(c0, _G0)]],
                gbuf.at[b, pl.ds(0, _G0)],
                gsems[b],
            ),
            pltpu.make_async_copy(
                table_hbm.at[idx_v.at[pl.ds(c0 + _G0, _G1)]],
                gbuf.at[b, pl.ds(_G0, _G1)],
                gsems[b],
            ),
        )

    def scatter_desc(i, b):
        return pltpu.make_async_copy(
            obuf.at[b],
            out_hbm.at[pl.ds(base + i * _CH, _CH)],
            ssems[b],
        )

    for b in range(_NBUF):
        for d in gather_descs(b, b):
            d.start()

    def outer(gi, carry):
        for b in range(_NBUF):
            i = gi * _NBUF + b
            for d in gather_descs(i, b):
                d.wait()

            @pl.when(gi > 0)
            def _():
                scatter_desc(i - _NBUF, b).wait()

            @plsc.parallel_loop(0, _CH, 1, unroll=8)
            def _(r):
                for v in range(_NVEC):
                    sl = pl.ds(v * _LANES, _LANES)
                    obuf[b, r, sl] = gbuf[b, r, sl] * 8.0 + pe_v[r, sl]
            scatter_desc(i, b).start()

            @pl.when(gi < _NOUTER - 1)
            def _():
                for d in gather_descs(i + _NBUF, b):
                    d.start()

        return carry

    lax.fori_loop(0, _NOUTER, outer, 0)

    for b in range(_NBUF):
        scatter_desc(_NCHUNK - _NBUF + b, b).wait()


def kernel(input, table):
    idx = input.reshape(-1).astype(jnp.int32)
    pe = _positional_rows()
    out = _emb_kernel(table, idx, pe)
    return out.reshape(_BATCH, _SEQ, _SIZE)


# speedup vs baseline: 1.0132x; 1.0132x over previous
import functools

import jax
import jax.numpy as jnp
from jax import lax
from jax.experimental import pallas as pl
from jax.experimental.pallas import tpu as pltpu
from jax.experimental.pallas import tpu_sc as plsc

_VOCAB = 1_000_000
_SIZE = 64
_BATCH = 4096
_SEQ = 200
_NC = 2
_NS = 16
_NW = _NC * _NS
_ROWS = _BATCH * _SEQ
_RPW = _ROWS // _NW
_CH = _SEQ
_NCHUNK = _RPW // _CH
_NBUF = 4
_NOB = 2
_NOUTER = _NCHUNK // _NBUF
_LANES = 16
_NVEC = _SIZE // _LANES
_G0 = 128
_G1 = _CH - _G0


def _positional_rows():
    pos = jnp.arange(_SEQ, dtype=jnp.float32)[:, None]
    period = jnp.power(10000.0, 2.0 * jnp.arange(_SIZE // 2, dtype=jnp.float32) / _SIZE)
    sin = jnp.sin(pos / period[None, :])
    cos = jnp.cos(pos / period[None, :])
    pe = jnp.zeros((_SEQ, _SIZE), dtype=jnp.float32)
    pe = pe.at[:, 0::2].set(sin)
    pe = pe.at[:, 1::2].set(cos)
    return pe


_mesh = plsc.VectorSubcoreMesh(core_axis_name="c", subcore_axis_name="s")


@functools.partial(
    pl.kernel,
    out_type=jax.ShapeDtypeStruct((_ROWS, _SIZE), jnp.float32),
    mesh=_mesh,
    scratch_types=[
        pltpu.VMEM((_RPW,), jnp.int32),
        pltpu.VMEM((_SEQ, _SIZE), jnp.float32),
        pltpu.VMEM((_NBUF, _CH, _SIZE), jnp.float32),
        pltpu.VMEM((_NOB, _CH, _SIZE), jnp.float32),
        pltpu.SemaphoreType.DMA,
        pltpu.SemaphoreType.DMA,
        pltpu.SemaphoreType.DMA,
        pltpu.SemaphoreType.DMA,
        pltpu.SemaphoreType.DMA,
        pltpu.SemaphoreType.DMA,
    ],
    compiler_params=pltpu.CompilerParams(use_tc_tiling_on_sc=False),
)
def _emb_kernel(table_hbm, idx_hbm, pe_hbm, out_hbm,
                idx_v, pe_v, gbuf, obuf,
                gsem0, gsem1, gsem2, gsem3, ssem0, ssem1):
    wid = lax.axis_index("s") * _NC + lax.axis_index("c")
    base = wid * _RPW
    gsems = (gsem0, gsem1, gsem2, gsem3)
    ssems = (ssem0, ssem1)

    pltpu.sync_copy(idx_hbm.at[pl.ds(base, _RPW)], idx_v)
    pltpu.sync_copy(pe_hbm, pe_v)

    def gather_descs(i, b):
        c0 = i * _CH
        return (
            pltpu.make_async_copy(
                table_hbm.at[idx_v.at[pl.ds(c0, _G0)]],
                gbuf.at[b, pl.ds(0, _G0)],
                gsems[b],
            ),
            pltpu.make_async_copy(
                table_hbm.at[idx_v.at[pl.ds(c0 + _G0, _G1)]],
                gbuf.at[b, pl.ds(_G0, _G1)],
                gsems[b],
            ),
        )

    def scatter_desc(i, ob):
        return pltpu.make_async_copy(
            obuf.at[ob],
            out_hbm.at[pl.ds(base + i * _CH, _CH)],
            ssems[ob],
        )

    for b in range(_NBUF):
        for d in gather_descs(b, b):
            d.start()

    def outer(gi, carry):
        for b in range(_NBUF):
            i = gi * _NBUF + b
            ob = b % _NOB
            for d in gather_descs(i, b):
                d.wait()

            if b >= _NOB:
                scatter_desc(i - _NOB, ob).wait()
            else:
                @pl.when(gi > 0)
                def _():
                    scatter_desc(i - _NOB, ob).wait()

            @plsc.parallel_loop(0, _CH, 1, unroll=8)
            def _(r):
                for v in range(_NVEC):
                    sl = pl.ds(v * _LANES, _LANES)
                    obuf[ob, r, sl] = gbuf[b, r, sl] * 8.0 + pe_v[r, sl]
            scatter_desc(i, ob).start()

            @pl.when(gi < _NOUTER - 1)
            def _():
                for d in gather_descs(i + _NBUF, b):
                    d.start()

        return carry

    lax.fori_loop(0, _NOUTER, outer, 0)

    for i in range(_NCHUNK - _NOB, _NCHUNK):
        scatter_desc(i, i % _NOB).wait()


def kernel(input, table):
    idx = input.reshape(-1).astype(jnp.int32)
    pe = _positional_rows()
    out = _emb_kernel(table, idx, pe)
    return out.reshape(_BATCH, _SEQ, _SIZE)
